# Initial kernel scaffold; baseline (speedup 1.0000x reference)
import functools

import jax
import jax.numpy as jnp
from jax import lax
from jax.experimental import pallas as pl
from jax.experimental.pallas import tpu as pltpu
from jax.experimental.pallas import tpu_sc as plsc

N = 10000
N_PAD = 10240
D = 128
E = 320000
NC = 2
NS = 16
K = 125
NCH = 80
ROWS_PT = N_PAD // NS


def _deg_body(dst_hbm, zeros_hbm, ones_hbm, deg_hbm, idx_v, ones_v, deg_sh, sem):
    del sem
    c = lax.axis_index("c")
    s = lax.axis_index("s")
    base = s * ROWS_PT
    pltpu.sync_copy(zeros_hbm, deg_sh.at[pl.ds(base, ROWS_PT)])
    pltpu.sync_copy(dst_hbm.at[c, s], idx_v)
    pltpu.sync_copy(ones_hbm, ones_v)
    plsc.subcore_barrier()

    def chunk(j, carry):
        pltpu.sync_copy(ones_v, deg_sh.at[idx_v.at[j]], add=True)
        return carry

    lax.fori_loop(0, NCH, chunk, 0)
    plsc.subcore_barrier()
    pltpu.sync_copy(deg_sh.at[pl.ds(base, ROWS_PT)],
                    deg_hbm.at[c, pl.ds(base, ROWS_PT)])


def _make_deg_kernel():
    mesh = plsc.VectorSubcoreMesh(core_axis_name="c", subcore_axis_name="s")
    return pl.kernel(
        _deg_body,
        out_type=jax.ShapeDtypeStruct((NC, N_PAD), jnp.float32),
        mesh=mesh,
        scratch_types=[
            pltpu.VMEM((NCH, K), jnp.int32),
            pltpu.VMEM((K,), jnp.float32),
            pltpu.VMEM_SHARED((N_PAD,), jnp.float32),
            pltpu.SemaphoreType.DMA,
        ],
    )


def _scat_body(src_hbm, dst_hbm, hs_hbm, zrows_hbm, s_hbm,
               sidx_v, didx_v, rows0, rows1, acc_sh, gsem0, gsem1):
    c = lax.axis_index("c")
    s = lax.axis_index("s")
    base = s * ROWS_PT
    pltpu.sync_copy(zrows_hbm, acc_sh.at[pl.ds(base, ROWS_PT)])
    pltpu.sync_copy(src_hbm.at[c, s], sidx_v)
    pltpu.sync_copy(dst_hbm.at[c, s], didx_v)
    plsc.subcore_barrier()

    pltpu.async_copy(hs_hbm.at[sidx_v.at[0]], rows0, gsem0)
    pltpu.async_copy(hs_hbm.at[sidx_v.at[1]], rows1, gsem1)

    def outer(g, carry):
        for b, (rows, gsem) in enumerate(((rows0, gsem0), (rows1, gsem1))):
            j = 2 * g + b
            pltpu.make_async_copy(hs_hbm.at[sidx_v.at[j]], rows, gsem).wait()
            pltpu.sync_copy(rows, acc_sh.at[didx_v.at[j]], add=True)

            @pl.when(g < NCH // 2 - 1)
            def _():
                pltpu.async_copy(hs_hbm.at[sidx_v.at[j + 2]], rows, gsem)

        return carry

    lax.fori_loop(0, NCH // 2, outer, 0)
    plsc.subcore_barrier()
    pltpu.sync_copy(acc_sh.at[pl.ds(base, ROWS_PT)],
                    s_hbm.at[c, pl.ds(base, ROWS_PT)])


def _make_scat_kernel():
    mesh = plsc.VectorSubcoreMesh(core_axis_name="c", subcore_axis_name="s")
    return pl.kernel(
        _scat_body,
        out_type=jax.ShapeDtypeStruct((NC, N_PAD, D), jnp.float32),
        mesh=mesh,
        scratch_types=[
            pltpu.VMEM((NCH, K), jnp.int32),
            pltpu.VMEM((NCH, K), jnp.int32),
            pltpu.VMEM((K, D), jnp.float32),
            pltpu.VMEM((K, D), jnp.float32),
            pltpu.VMEM_SHARED((N_PAD, D), jnp.float32),
            pltpu.SemaphoreType.DMA,
            pltpu.SemaphoreType.DMA,
        ],
    )


_RB = 1024


def _lin_body(x_ref, w_ref, deg_ref, hs_ref, dinv_ref):
    h = jnp.dot(x_ref[...], w_ref[...], preferred_element_type=jnp.float32)
    d = deg_ref[...]
    dinv = lax.rsqrt(d[0] + d[1] + 1.0)
    hs_ref[...] = h * dinv
    dinv_ref[...] = dinv


def _make_lin_kernel():
    return pl.pallas_call(
        _lin_body,
        grid=(N_PAD // _RB,),
        in_specs=[
            pl.BlockSpec((_RB, D), lambda i: (i, 0)),
            pl.BlockSpec((D, D), lambda i: (0, 0)),
            pl.BlockSpec((NC, _RB, 1), lambda i: (0, i, 0)),
        ],
        out_specs=[
            pl.BlockSpec((_RB, D), lambda i: (i, 0)),
            pl.BlockSpec((_RB, 1), lambda i: (i, 0)),
        ],
        out_shape=[
            jax.ShapeDtypeStruct((N_PAD, D), jnp.float32),
            jax.ShapeDtypeStruct((N_PAD, 1), jnp.float32),
        ],
    )


def _ep_body(s_ref, hs_ref, dinv_ref, b_ref, out_ref):
    sacc = s_ref[...]
    acc = sacc[0] + sacc[1] + hs_ref[...]
    out_ref[...] = jnp.maximum(acc * dinv_ref[...] + b_ref[...], 0.0)


def _make_ep_kernel():
    return pl.pallas_call(
        _ep_body,
        grid=(N_PAD // _RB,),
        in_specs=[
            pl.BlockSpec((NC, _RB, D), lambda i: (0, i, 0)),
            pl.BlockSpec((_RB, D), lambda i: (i, 0)),
            pl.BlockSpec((_RB, 1), lambda i: (i, 0)),
            pl.BlockSpec((1, D), lambda i: (0, 0)),
        ],
        out_specs=pl.BlockSpec((_RB, D), lambda i: (i, 0)),
        out_shape=jax.ShapeDtypeStruct((N_PAD, D), jnp.float32),
    )


_deg_kernel = _make_deg_kernel()
_scat_kernel = _make_scat_kernel()
_lin_kernel = _make_lin_kernel()
_ep_kernel = _make_ep_kernel()


@jax.jit
def kernel(x, edge_index, W, b):
    src = edge_index[0].reshape(NC, NS, NCH, K)
    dst = edge_index[1].reshape(NC, NS, NCH, K)
    x_pad = jnp.pad(x, ((0, N_PAD - N), (0, 0)))
    zeros1 = jnp.zeros((ROWS_PT,), jnp.float32)
    zrows = jnp.zeros((ROWS_PT, D), jnp.float32)
    ones = jnp.ones((K,), jnp.float32)

    deg = _deg_kernel(dst, zeros1, ones)
    hs, dinv = _lin_kernel(x_pad, W, deg.reshape(NC, N_PAD, 1))
    S = _scat_kernel(src, dst, hs, zrows)
    out = _ep_kernel(S, hs, dinv, b.reshape(1, D))
    return out[:N]


# speedup vs baseline: 29.6847x; 29.6847x over previous
import functools

import jax
import jax.numpy as jnp
from jax import lax
from jax.experimental import pallas as pl
from jax.experimental.pallas import tpu as pltpu
from jax.experimental.pallas import tpu_sc as plsc

N = 10000
N_PAD = 10240
D = 128
E = 320000
NC = 2
NS = 16
K = 125
NCH = 80
ROWS_PT = N_PAD // NS


def _deg_body(dst_hbm, zeros_hbm, ones_hbm, deg_hbm, idx_v, ones_v, deg_sh, sem):
    del sem
    c = lax.axis_index("c")
    s = lax.axis_index("s")
    base = s * ROWS_PT
    pltpu.sync_copy(zeros_hbm, deg_sh.at[pl.ds(base, ROWS_PT)])
    pltpu.sync_copy(dst_hbm.at[c, s], idx_v)
    pltpu.sync_copy(ones_hbm, ones_v)
    plsc.subcore_barrier()

    def chunk(j, carry):
        pltpu.sync_copy(ones_v, deg_sh.at[idx_v.at[j]], add=True)
        return carry

    lax.fori_loop(0, NCH, chunk, 0)
    plsc.subcore_barrier()
    pltpu.sync_copy(deg_sh.at[pl.ds(base, ROWS_PT)],
                    deg_hbm.at[c, pl.ds(base, ROWS_PT)])


def _make_deg_kernel():
    mesh = plsc.VectorSubcoreMesh(core_axis_name="c", subcore_axis_name="s")
    return pl.kernel(
        _deg_body,
        out_type=jax.ShapeDtypeStruct((NC, N_PAD), jnp.float32),
        mesh=mesh,
        scratch_types=[
            pltpu.VMEM((NCH, K), jnp.int32),
            pltpu.VMEM((K,), jnp.float32),
            pltpu.VMEM_SHARED((N_PAD,), jnp.float32),
            pltpu.SemaphoreType.DMA,
        ],
    )


def _scat_body(src_hbm, dst_hbm, hs_hbm, zrows_hbm, s_hbm,
               sidx_v, didx_v, rows0, acc_sh, gsem0):
    c = lax.axis_index("c")
    s = lax.axis_index("s")
    base = s * ROWS_PT
    pltpu.sync_copy(zrows_hbm, acc_sh.at[pl.ds(base, ROWS_PT)])
    pltpu.sync_copy(src_hbm.at[c, s], sidx_v)
    pltpu.sync_copy(dst_hbm.at[c, s], didx_v)
    plsc.subcore_barrier()

    def chunk(j, carry):
        pltpu.async_copy(hs_hbm.at[sidx_v.at[j]], rows0, gsem0).wait()
        pltpu.sync_copy(rows0, acc_sh.at[didx_v.at[j]], add=True)
        return carry

    lax.fori_loop(0, NCH, chunk, 0)
    plsc.subcore_barrier()
    pltpu.sync_copy(acc_sh.at[pl.ds(base, ROWS_PT)],
                    s_hbm.at[c, pl.ds(base, ROWS_PT)])


def _make_scat_kernel():
    mesh = plsc.VectorSubcoreMesh(core_axis_name="c", subcore_axis_name="s")
    return pl.kernel(
        _scat_body,
        out_type=jax.ShapeDtypeStruct((NC, N_PAD, D), jnp.float32),
        mesh=mesh,
        scratch_types=[
            pltpu.VMEM((NCH, K), jnp.int32),
            pltpu.VMEM((NCH, K), jnp.int32),
            pltpu.VMEM((K, D), jnp.float32),
            pltpu.VMEM_SHARED((N_PAD, D), jnp.float32),
            pltpu.SemaphoreType.DMA,
        ],
    )


_RB = 1024


def _lin_body(x_ref, w_ref, deg_ref, hs_ref, dinv_ref):
    h = jnp.dot(x_ref[...], w_ref[...], preferred_element_type=jnp.float32)
    d = deg_ref[...]
    dinv = lax.rsqrt(d[0] + d[1] + 1.0)
    hs_ref[...] = h * dinv
    dinv_ref[...] = dinv


def _make_lin_kernel():
    return pl.pallas_call(
        _lin_body,
        grid=(N_PAD // _RB,),
        in_specs=[
            pl.BlockSpec((_RB, D), lambda i: (i, 0)),
            pl.BlockSpec((D, D), lambda i: (0, 0)),
            pl.BlockSpec((NC, _RB, 1), lambda i: (0, i, 0)),
        ],
        out_specs=[
            pl.BlockSpec((_RB, D), lambda i: (i, 0)),
            pl.BlockSpec((_RB, 1), lambda i: (i, 0)),
        ],
        out_shape=[
            jax.ShapeDtypeStruct((N_PAD, D), jnp.float32),
            jax.ShapeDtypeStruct((N_PAD, 1), jnp.float32),
        ],
    )


def _ep_body(s_ref, hs_ref, dinv_ref, b_ref, out_ref):
    sacc = s_ref[...]
    acc = sacc[0] + sacc[1] + hs_ref[...]
    out_ref[...] = jnp.maximum(acc * dinv_ref[...] + b_ref[...], 0.0)


def _make_ep_kernel():
    return pl.pallas_call(
        _ep_body,
        grid=(N_PAD // _RB,),
        in_specs=[
            pl.BlockSpec((NC, _RB, D), lambda i: (0, i, 0)),
            pl.BlockSpec((_RB, D), lambda i: (i, 0)),
            pl.BlockSpec((_RB, 1), lambda i: (i, 0)),
            pl.BlockSpec((1, D), lambda i: (0, 0)),
        ],
        out_specs=pl.BlockSpec((_RB, D), lambda i: (i, 0)),
        out_shape=jax.ShapeDtypeStruct((N_PAD, D), jnp.float32),
    )


_deg_kernel = _make_deg_kernel()
_scat_kernel = _make_scat_kernel()
_lin_kernel = _make_lin_kernel()
_ep_kernel = _make_ep_kernel()


@jax.jit
def kernel(x, edge_index, W, b):
    src = edge_index[0].reshape(NC, NS, NCH, K)
    dst = edge_index[1].reshape(NC, NS, NCH, K)
    x_pad = jnp.pad(x, ((0, N_PAD - N), (0, 0)))
    zeros1 = jnp.zeros((ROWS_PT,), jnp.float32)
    zrows = jnp.zeros((ROWS_PT, D), jnp.float32)
    ones = jnp.ones((K,), jnp.float32)

    deg = _deg_kernel(dst, zeros1, ones)
    hs, dinv = _lin_kernel(x_pad, W, deg.reshape(NC, N_PAD, 1))
    S = _scat_kernel(src, dst, hs, zrows)
    out = _ep_kernel(S, hs, dinv, b.reshape(1, D))
    return out[:N]


# speedup vs baseline: 40.8459x; 1.3760x over previous
import functools

import jax
import jax.numpy as jnp
from jax import lax
from jax.experimental import pallas as pl
from jax.experimental.pallas import tpu as pltpu
from jax.experimental.pallas import tpu_sc as plsc

N = 10000
N_PAD = 10240
D = 128
E = 320000
NC = 2
NS = 16
NW = NC * NS
K = 128
NCH = 80
EPW = NCH * K
E_PAD = NW * EPW
ROWS_PT = N_PAD // NS


def _deg_body(dst_hbm, zeros_hbm, ones_hbm, deg_hbm, idx_v, ones_v, deg_sh, sem):
    del sem
    c = lax.axis_index("c")
    s = lax.axis_index("s")
    base = s * ROWS_PT
    pltpu.sync_copy(zeros_hbm, deg_sh.at[pl.ds(base, ROWS_PT)])
    pltpu.sync_copy(dst_hbm.at[c, s], idx_v)
    pltpu.sync_copy(ones_hbm, ones_v)
    plsc.subcore_barrier()

    def chunk(j, carry):
        pltpu.sync_copy(ones_v, deg_sh.at[idx_v.at[j]], add=True)
        return carry

    lax.fori_loop(0, NCH, chunk, 0)
    plsc.subcore_barrier()
    pltpu.sync_copy(deg_sh.at[pl.ds(base, ROWS_PT)],
                    deg_hbm.at[c, pl.ds(base, ROWS_PT)])


def _make_deg_kernel():
    mesh = plsc.VectorSubcoreMesh(core_axis_name="c", subcore_axis_name="s")
    return pl.kernel(
        _deg_body,
        out_type=jax.ShapeDtypeStruct((NC, N_PAD), jnp.float32),
        mesh=mesh,
        scratch_types=[
            pltpu.VMEM((NCH, K), jnp.int32),
            pltpu.VMEM((K,), jnp.float32),
            pltpu.VMEM_SHARED((N_PAD,), jnp.float32),
            pltpu.SemaphoreType.DMA,
        ],
    )


def _parity(sel, fn0, fn1):
    @pl.when(sel == 0)
    def _():
        fn0()

    @pl.when(sel == 1)
    def _():
        fn1()


def _scat_body(src_hbm, dst_hbm, hs_hbm, zrows_hbm, s_hbm,
               iv, rows_v, acc_sh, isem0, isem1, gsem0, gsem1):
    c = lax.axis_index("c")
    s = lax.axis_index("s")
    base = s * ROWS_PT
    pltpu.sync_copy(zrows_hbm, acc_sh.at[pl.ds(base, ROWS_PT)])
    plsc.subcore_barrier()

    def step(j, carry):
        s3 = lax.rem(j, 3)
        p = lax.rem(j, 2)

        @pl.when(j < NCH)
        def _():
            def issue(isem):
                pltpu.async_copy(src_hbm.at[c, s, j], iv.at[s3, 0], isem)
                pltpu.async_copy(dst_hbm.at[c, s, j], iv.at[s3, 1], isem)

            _parity(p, lambda: issue(isem0), lambda: issue(isem1))

        @pl.when(jnp.logical_and(j >= 1, j <= NCH))
        def _():
            jm = j - 1
            m3 = lax.rem(jm, 3)
            mp = lax.rem(jm, 2)

            def drain(isem):
                pltpu.make_async_copy(src_hbm.at[c, s, 0], iv.at[m3, 0], isem).wait()
                pltpu.make_async_copy(dst_hbm.at[c, s, 0], iv.at[m3, 1], isem).wait()

            _parity(mp, lambda: drain(isem0), lambda: drain(isem1))

            def gath(gsem):
                pltpu.async_copy(hs_hbm.at[iv.at[m3, 0]], rows_v.at[mp], gsem)

            _parity(mp, lambda: gath(gsem0), lambda: gath(gsem1))

        @pl.when(j >= 2)
        def _():
            jm = j - 2
            m3 = lax.rem(jm, 3)
            mp = lax.rem(jm, 2)

            def drain(gsem):
                pltpu.make_async_copy(hs_hbm.at[iv.at[m3, 0]], rows_v.at[mp], gsem).wait()

            _parity(mp, lambda: drain(gsem0), lambda: drain(gsem1))
            pltpu.sync_copy(rows_v.at[mp], acc_sh.at[iv.at[m3, 1]], add=True)

        return carry

    lax.fori_loop(0, NCH + 2, step, 0)
    plsc.subcore_barrier()
    pltpu.sync_copy(acc_sh.at[pl.ds(base, ROWS_PT)],
                    s_hbm.at[c, pl.ds(base, ROWS_PT)])


def _make_scat_kernel():
    mesh = plsc.VectorSubcoreMesh(core_axis_name="c", subcore_axis_name="s")
    return pl.kernel(
        _scat_body,
        out_type=jax.ShapeDtypeStruct((NC, N_PAD, D), jnp.float32),
        mesh=mesh,
        scratch_types=[
            pltpu.VMEM((3, 2, K), jnp.int32),
            pltpu.VMEM((2, K, D), jnp.float32),
            pltpu.VMEM_SHARED((N_PAD, D), jnp.float32),
            pltpu.SemaphoreType.DMA,
            pltpu.SemaphoreType.DMA,
            pltpu.SemaphoreType.DMA,
            pltpu.SemaphoreType.DMA,
        ],
    )


_RB = 1024


def _lin_body(x_ref, w_ref, deg_ref, hs_ref, dinv_ref):
    h = jnp.dot(x_ref[...], w_ref[...], preferred_element_type=jnp.float32)
    d = deg_ref[...]
    dinv = lax.rsqrt(d[0] + d[1] + 1.0)
    hs_ref[...] = h * dinv
    dinv_ref[...] = dinv


def _make_lin_kernel():
    return pl.pallas_call(
        _lin_body,
        grid=(N_PAD // _RB,),
        in_specs=[
            pl.BlockSpec((_RB, D), lambda i: (i, 0)),
            pl.BlockSpec((D, D), lambda i: (0, 0)),
            pl.BlockSpec((NC, _RB, 1), lambda i: (0, i, 0)),
        ],
        out_specs=[
            pl.BlockSpec((_RB, D), lambda i: (i, 0)),
            pl.BlockSpec((_RB, 1), lambda i: (i, 0)),
        ],
        out_shape=[
            jax.ShapeDtypeStruct((N_PAD, D), jnp.float32),
            jax.ShapeDtypeStruct((N_PAD, 1), jnp.float32),
        ],
    )


def _ep_body(s_ref, hs_ref, dinv_ref, b_ref, out_ref):
    sacc = s_ref[...]
    acc = sacc[0] + sacc[1] + hs_ref[...]
    out_ref[...] = jnp.maximum(acc * dinv_ref[...] + b_ref[...], 0.0)


def _make_ep_kernel():
    return pl.pallas_call(
        _ep_body,
        grid=(N_PAD // _RB,),
        in_specs=[
            pl.BlockSpec((NC, _RB, D), lambda i: (0, i, 0)),
            pl.BlockSpec((_RB, D), lambda i: (i, 0)),
            pl.BlockSpec((_RB, 1), lambda i: (i, 0)),
            pl.BlockSpec((1, D), lambda i: (0, 0)),
        ],
        out_specs=pl.BlockSpec((_RB, D), lambda i: (i, 0)),
        out_shape=jax.ShapeDtypeStruct((N_PAD, D), jnp.float32),
    )


_deg_kernel = _make_deg_kernel()
_scat_kernel = _make_scat_kernel()
_lin_kernel = _make_lin_kernel()
_ep_kernel = _make_ep_kernel()


@jax.jit
def kernel(x, edge_index, W, b):
    n_dummy = E_PAD - E
    pad_idx = (N + jax.lax.rem(jnp.arange(n_dummy, dtype=jnp.int32),
                               jnp.int32(N_PAD - N)))
    src = jnp.concatenate([edge_index[0], pad_idx]).reshape(NC, NS, NCH, K)
    dst = jnp.concatenate([edge_index[1], pad_idx]).reshape(NC, NS, NCH, K)
    x_pad = jnp.pad(x, ((0, N_PAD - N), (0, 0)))
    zeros1 = jnp.zeros((ROWS_PT,), jnp.float32)
    zrows = jnp.zeros((ROWS_PT, D), jnp.float32)
    ones = jnp.ones((K,), jnp.float32)

    deg = _deg_kernel(dst, zeros1, ones)
    hs, dinv = _lin_kernel(x_pad, W, deg.reshape(NC, N_PAD, 1))
    S = _scat_kernel(src, dst, hs, zrows)
    out = _ep_kernel(S, hs, dinv, b.reshape(1, D))
    return out[:N]


# speedup vs baseline: 42.8507x; 1.0491x over previous
import functools

import jax
import jax.numpy as jnp
from jax import lax
from jax.experimental import pallas as pl
from jax.experimental.pallas import tpu as pltpu
from jax.experimental.pallas import tpu_sc as plsc

N = 10000
N_PAD = 10240
D = 128
E = 320000
NC = 2
NS = 16
NW = NC * NS
K = 96
NCH = 105
EPW = NCH * K
E_PAD = NW * EPW
ROWS_PT = N_PAD // NS


def _deg_body(dst_hbm, zeros_hbm, ones_hbm, deg_hbm, idx_v, ones_v, deg_sh, sem):
    del sem
    c = lax.axis_index("c")
    s = lax.axis_index("s")
    base = s * ROWS_PT
    pltpu.sync_copy(zeros_hbm, deg_sh.at[pl.ds(base, ROWS_PT)])
    pltpu.sync_copy(dst_hbm.at[c, s], idx_v)
    pltpu.sync_copy(ones_hbm, ones_v)
    plsc.subcore_barrier()

    def chunk(j, carry):
        pltpu.sync_copy(ones_v, deg_sh.at[idx_v.at[j]], add=True)
        return carry

    lax.fori_loop(0, NCH, chunk, 0)
    plsc.subcore_barrier()
    pltpu.sync_copy(deg_sh.at[pl.ds(base, ROWS_PT)],
                    deg_hbm.at[c, pl.ds(base, ROWS_PT)])


def _make_deg_kernel():
    mesh = plsc.VectorSubcoreMesh(core_axis_name="c", subcore_axis_name="s")
    return pl.kernel(
        _deg_body,
        out_type=jax.ShapeDtypeStruct((NC, N_PAD), jnp.float32),
        mesh=mesh,
        scratch_types=[
            pltpu.VMEM((NCH, K), jnp.int32),
            pltpu.VMEM((K,), jnp.float32),
            pltpu.VMEM_SHARED((N_PAD,), jnp.float32),
            pltpu.SemaphoreType.DMA,
        ],
    )


def _parity(sel, fn0, fn1):
    @pl.when(sel == 0)
    def _():
        fn0()

    @pl.when(sel == 1)
    def _():
        fn1()


def _mod3(sel, fn, sem0, sem1, sem2):
    @pl.when(sel == 0)
    def _():
        fn(sem0)

    @pl.when(sel == 1)
    def _():
        fn(sem1)

    @pl.when(sel == 2)
    def _():
        fn(sem2)


def _scat_body(src_hbm, dst_hbm, hs_hbm, zrows_hbm, s_hbm,
               iv, rows_v, acc_sh, isem0, isem1, gsem0, gsem1, gsem2):
    c = lax.axis_index("c")
    s = lax.axis_index("s")
    base = s * ROWS_PT
    pltpu.sync_copy(zrows_hbm, acc_sh.at[pl.ds(base, ROWS_PT)])
    plsc.subcore_barrier()

    def step(j, carry):
        s5 = lax.rem(j, 5)
        p = lax.rem(j, 2)

        @pl.when(j < NCH)
        def _():
            def issue(isem):
                pltpu.async_copy(src_hbm.at[c, s, j], iv.at[s5, 0], isem)
                pltpu.async_copy(dst_hbm.at[c, s, j], iv.at[s5, 1], isem)

            _parity(p, lambda: issue(isem0), lambda: issue(isem1))

        @pl.when(jnp.logical_and(j >= 1, j <= NCH))
        def _():
            jm = j - 1
            m5 = lax.rem(jm, 5)
            mp = lax.rem(jm, 2)
            m3 = lax.rem(jm, 3)

            def drain(isem):
                pltpu.make_async_copy(src_hbm.at[c, s, 0], iv.at[m5, 0], isem).wait()
                pltpu.make_async_copy(dst_hbm.at[c, s, 0], iv.at[m5, 1], isem).wait()

            _parity(mp, lambda: drain(isem0), lambda: drain(isem1))

            def gath(gsem):
                pltpu.async_copy(hs_hbm.at[iv.at[m5, 0]], rows_v.at[m3], gsem)

            _mod3(m3, gath, gsem0, gsem1, gsem2)

        @pl.when(j >= 3)
        def _():
            jm = j - 3
            m5 = lax.rem(jm, 5)
            m3 = lax.rem(jm, 3)

            def drain(gsem):
                pltpu.make_async_copy(hs_hbm.at[iv.at[m5, 0]], rows_v.at[m3], gsem).wait()

            _mod3(m3, drain, gsem0, gsem1, gsem2)
            pltpu.sync_copy(rows_v.at[m3], acc_sh.at[iv.at[m5, 1]], add=True)

        return carry

    lax.fori_loop(0, NCH + 3, step, 0)
    plsc.subcore_barrier()
    pltpu.sync_copy(acc_sh.at[pl.ds(base, ROWS_PT)],
                    s_hbm.at[c, pl.ds(base, ROWS_PT)])


def _make_scat_kernel():
    mesh = plsc.VectorSubcoreMesh(core_axis_name="c", subcore_axis_name="s")
    return pl.kernel(
        _scat_body,
        out_type=jax.ShapeDtypeStruct((NC, N_PAD, D), jnp.float32),
        mesh=mesh,
        scratch_types=[
            pltpu.VMEM((5, 2, K), jnp.int32),
            pltpu.VMEM((3, K, D), jnp.float32),
            pltpu.VMEM_SHARED((N_PAD, D), jnp.float32),
            pltpu.SemaphoreType.DMA,
            pltpu.SemaphoreType.DMA,
            pltpu.SemaphoreType.DMA,
            pltpu.SemaphoreType.DMA,
            pltpu.SemaphoreType.DMA,
        ],
    )


_RB = 1024


def _lin_body(x_ref, w_ref, deg_ref, hs_ref, dinv_ref):
    h = jnp.dot(x_ref[...], w_ref[...], preferred_element_type=jnp.float32)
    d = deg_ref[...]
    dinv = lax.rsqrt(d[0] + d[1] + 1.0)
    hs_ref[...] = h * dinv
    dinv_ref[...] = dinv


def _make_lin_kernel():
    return pl.pallas_call(
        _lin_body,
        grid=(N_PAD // _RB,),
        in_specs=[
            pl.BlockSpec((_RB, D), lambda i: (i, 0)),
            pl.BlockSpec((D, D), lambda i: (0, 0)),
            pl.BlockSpec((NC, _RB, 1), lambda i: (0, i, 0)),
        ],
        out_specs=[
            pl.BlockSpec((_RB, D), lambda i: (i, 0)),
            pl.BlockSpec((_RB, 1), lambda i: (i, 0)),
        ],
        out_shape=[
            jax.ShapeDtypeStruct((N_PAD, D), jnp.float32),
            jax.ShapeDtypeStruct((N_PAD, 1), jnp.float32),
        ],
    )


def _ep_body(s_ref, hs_ref, dinv_ref, b_ref, out_ref):
    sacc = s_ref[...]
    acc = sacc[0] + sacc[1] + hs_ref[...]
    out_ref[...] = jnp.maximum(acc * dinv_ref[...] + b_ref[...], 0.0)


def _make_ep_kernel():
    return pl.pallas_call(
        _ep_body,
        grid=(N_PAD // _RB,),
        in_specs=[
            pl.BlockSpec((NC, _RB, D), lambda i: (0, i, 0)),
            pl.BlockSpec((_RB, D), lambda i: (i, 0)),
            pl.BlockSpec((_RB, 1), lambda i: (i, 0)),
            pl.BlockSpec((1, D), lambda i: (0, 0)),
        ],
        out_specs=pl.BlockSpec((_RB, D), lambda i: (i, 0)),
        out_shape=jax.ShapeDtypeStruct((N_PAD, D), jnp.float32),
    )


_deg_kernel = _make_deg_kernel()
_scat_kernel = _make_scat_kernel()
_lin_kernel = _make_lin_kernel()
_ep_kernel = _make_ep_kernel()


@jax.jit
def kernel(x, edge_index, W, b):
    n_dummy = E_PAD - E
    pad_idx = (N + jax.lax.rem(jnp.arange(n_dummy, dtype=jnp.int32),
                               jnp.int32(N_PAD - N)))
    src = jnp.concatenate([edge_index[0], pad_idx]).reshape(NC, NS, NCH, K)
    dst = jnp.concatenate([edge_index[1], pad_idx]).reshape(NC, NS, NCH, K)
    x_pad = jnp.pad(x, ((0, N_PAD - N), (0, 0)))
    zeros1 = jnp.zeros((ROWS_PT,), jnp.float32)
    zrows = jnp.zeros((ROWS_PT, D), jnp.float32)
    ones = jnp.ones((K,), jnp.float32)

    deg = _deg_kernel(dst, zeros1, ones)
    hs, dinv = _lin_kernel(x_pad, W, deg.reshape(NC, N_PAD, 1))
    S = _scat_kernel(src, dst, hs, zrows)
    out = _ep_kernel(S, hs, dinv, b.reshape(1, D))
    return out[:N]
